# baseline (speedup 1.0000x reference)
import functools
import math

import jax
import jax.numpy as jnp
from jax import lax
from jax.experimental import pallas as pl
from jax.experimental.pallas import tpu as pltpu
from jax.experimental.pallas import tpu_sc as plsc

VOCAB = 1000000
D = 64
B = 4096
L = 200
N = B * L

NC = 2
NS = 16
NW = NC * NS
B_PER_W = N // NW

CHUNK = 128
K = 8
SUPER = CHUNK * K
N_SUPER = B_PER_W // SUPER

SCALE = math.sqrt(D)

_mesh = plsc.VectorSubcoreMesh(core_axis_name="c", subcore_axis_name="s")


@functools.partial(
    pl.kernel,
    mesh=_mesh,
    out_type=jax.ShapeDtypeStruct((N, D), jnp.float32),
    scratch_types=[
        pltpu.VMEM((SUPER,), jnp.int32),
        pltpu.VMEM((SUPER, D), jnp.float32),
        pltpu.SemaphoreType.DMA,
    ],
    compiler_params=pltpu.CompilerParams(use_tc_tiling_on_sc=False),
)
def _emb_lookup(x_hbm, table_hbm, out_hbm, idx_v, rows_v, sem):
    wid = lax.axis_index("s") * NC + lax.axis_index("c")
    base = wid * B_PER_W

    def super_body(g, carry):
        off = base + g * SUPER
        pltpu.sync_copy(x_hbm.at[pl.ds(off, SUPER)], idx_v)
        handles = []
        for j in range(K):
            handles.append(
                pltpu.async_copy(
                    table_hbm.at[idx_v.at[pl.ds(j * CHUNK, CHUNK)]],
                    rows_v.at[pl.ds(j * CHUNK, CHUNK)],
                    sem,
                )
            )
        for h in handles:
            h.wait()

        def mul_body(r, c2):
            for c in range(D // 16):
                rows_v[r, pl.ds(c * 16, 16)] = (
                    rows_v[r, pl.ds(c * 16, 16)] * jnp.float32(SCALE)
                )
            return c2

        lax.fori_loop(0, SUPER, mul_body, 0)

        pltpu.sync_copy(rows_v, out_hbm.at[pl.ds(off, SUPER)])
        return carry

    lax.fori_loop(0, N_SUPER, super_body, 0)


def kernel(x, table):
    out = _emb_lookup(x.reshape(N).astype(jnp.int32), table)
    return out.reshape(B, L, D)


# speedup vs baseline: 1.1107x; 1.1107x over previous
import functools
import math

import jax
import jax.numpy as jnp
from jax import lax
from jax.experimental import pallas as pl
from jax.experimental.pallas import tpu as pltpu
from jax.experimental.pallas import tpu_sc as plsc

VOCAB = 1000000
D = 64
B = 4096
L = 200
N = B * L

NC = 2
NS = 16
NW = NC * NS
B_PER_W = N // NW

GATHER = 128
G_PER_CHUNK = 2
CHUNK = GATHER * G_PER_CHUNK
N_CHUNKS = B_PER_W // CHUNK
NBUF = 5

SCALE = math.sqrt(D)

_mesh = plsc.VectorSubcoreMesh(core_axis_name="c", subcore_axis_name="s")


@functools.partial(
    pl.kernel,
    mesh=_mesh,
    out_type=jax.ShapeDtypeStruct((N, D), jnp.float32),
    scratch_types=[
        pltpu.VMEM((B_PER_W,), jnp.int32),
        pltpu.VMEM((NBUF, CHUNK, D), jnp.float32),
    ]
    + [pltpu.SemaphoreType.DMA] * (2 * NBUF),
    compiler_params=pltpu.CompilerParams(use_tc_tiling_on_sc=False),
)
def _emb_lookup(x_hbm, table_hbm, out_hbm, idx_v, rows_v, *sems):
    gsems = sems[:NBUF]
    osems = sems[NBUF:]
    wid = lax.axis_index("s") * NC + lax.axis_index("c")
    base = wid * B_PER_W

    pltpu.sync_copy(x_hbm.at[pl.ds(base, B_PER_W)], idx_v)

    def g_copies(c, b):
        return [
            pltpu.make_async_copy(
                table_hbm.at[idx_v.at[pl.ds(c * CHUNK + j * GATHER, GATHER)]],
                rows_v.at[b, pl.ds(j * GATHER, GATHER)],
                gsems[b],
            )
            for j in range(G_PER_CHUNK)
        ]

    def w_copy(c, b):
        return pltpu.make_async_copy(
            rows_v.at[b],
            out_hbm.at[pl.ds(base + c * CHUNK, CHUNK)],
            osems[b],
        )

    def scale_buf(b):
        @plsc.parallel_loop(0, CHUNK, 1, unroll=8)
        def _scale(r):
            for cc in range(D // 16):
                sl = pl.ds(cc * 16, 16)
                rows_v[b, r, sl] = rows_v[b, r, sl] * jnp.float32(SCALE)

    def step(c, par, fire, drain_wb):
        b = par
        nb = (par + NBUF - 1) % NBUF
        for cp in g_copies(c, b):
            cp.wait()
        scale_buf(b)
        w_copy(c, b).start()
        if fire:
            if drain_wb:
                w_copy(c - 1, nb).wait()
            for cp in g_copies(c + NBUF - 1, nb):
                cp.start()

    for c0 in range(NBUF - 1):
        for cp in g_copies(c0, c0):
            cp.start()

    step(0, 0, fire=True, drain_wb=False)
    for par in range(1, NBUF):
        step(par, par, fire=True, drain_wb=True)

    def outer(go, carry):
        for par in range(NBUF):
            step(go * NBUF + par, par, fire=True, drain_wb=True)
        return carry

    lax.fori_loop(1, N_CHUNKS // NBUF - 1, outer, 0)

    for par in range(NBUF):
        c = (N_CHUNKS - NBUF) + par
        step(c, par, fire=(par == 0), drain_wb=(par == 0))

    for par in range(NBUF):
        w_copy(N_CHUNKS - NBUF + par, par).wait()


def kernel(x, table):
    out = _emb_lookup(x.reshape(N).astype(jnp.int32), table)
    return out.reshape(B, L, D)
